# baseline (speedup 1.0000x reference)
import math

import jax
import jax.numpy as jnp
import numpy as np
from jax.experimental import pallas as pl

MAX_LEN = 8192
DIM = 1024
SCALE = math.sqrt(DIM)

ROWS_PER_BLOCK = 256


def _make_pe(max_len, dim):
    position = np.arange(max_len, dtype=np.float64)[:, None]
    div_term = np.exp(
        np.arange(0, dim, 2, dtype=np.float64) * -(math.log(10000.0) / dim)
    )
    pe = np.zeros((max_len, dim), dtype=np.float32)
    pe[:, 0::2] = np.sin(position * div_term).astype(np.float32)
    pe[:, 1::2] = np.cos(position * div_term).astype(np.float32)
    return pe


_PE = jnp.asarray(_make_pe(MAX_LEN, DIM))


def _block_kernel(emb_ref, pe_ref, out_ref):
    pe = pe_ref[...]
    out_ref[...] = emb_ref[...] * SCALE + pe[:, None, :]


def kernel(emb):
    seq, feat, dim = emb.shape
    pe = _PE[:seq]
    grid = (seq // ROWS_PER_BLOCK,)
    return pl.pallas_call(
        _block_kernel,
        grid=grid,
        in_specs=[
            pl.BlockSpec((ROWS_PER_BLOCK, feat, dim), lambda i: (i, 0, 0)),
            pl.BlockSpec((ROWS_PER_BLOCK, dim), lambda i: (i, 0)),
        ],
        out_specs=pl.BlockSpec((ROWS_PER_BLOCK, feat, dim), lambda i: (i, 0, 0)),
        out_shape=jax.ShapeDtypeStruct((seq, feat, dim), emb.dtype),
    )(emb, pe)
